# baseline (speedup 1.0000x reference)
import functools

import jax
import jax.numpy as jnp
from jax import lax
from jax.experimental import pallas as pl
from jax.experimental.pallas import tpu as pltpu
from jax.experimental.pallas import tpu_sc as plsc

K = 4
B = 2
DEGREE = 8
LEVELS = [65536, 32768, 16384, 8192, 4096, 2048]
NC = 2
NS = 16
NW = NC * NS
ECHUNK = 2048
PCHUNK = 2048

K_CH = {65536: 1, 32768: 1, 16384: 2, 8192: 4, 4096: 8, 2048: 16}


def _mesh():
    return plsc.VectorSubcoreMesh(core_axis_name="c", subcore_axis_name="s",
                                  num_cores=NC, num_subcores=NS)


def _worker_id():
    return lax.axis_index("s") * NC + lax.axis_index("c")


@functools.cache
def _make_apply(R, n, step1):
    e = n * DEGREE
    half = n >= 65536
    H = 2 if half else 1
    nh = n // H
    k_ch = K_CH[n]
    assert R % k_ch == 0
    groups = R // k_ch
    items = groups * H
    rounds = -(-items // NW)

    def body(x_hbm, ed_hbm, w_hbm, prev_hbm, out_hbm,
             G, A, SB, WB, PB, sem):
        wid = _worker_id()
        for rnd in range(rounds):
            item = rnd * NW + wid

            @pl.when(item < items)
            def _():
                g = item // H
                h = item % H
                lo = h * nh
                rbase = g * k_ch
                for c in range(k_ch):
                    pltpu.sync_copy(x_hbm.at[pl.ds((rbase + c) * n, n)],
                                    G.at[pl.ds(c * n, n)])

                def zbody(j, carry):
                    zz = jnp.zeros((16,), jnp.float32)
                    for c in range(k_ch):
                        A[pl.ds(c * nh + j * 16, 16)] = zz
                    return carry
                lax.fori_loop(0, nh // 16, zbody, 0, unroll=8)

                def estart(ci, buf):
                    off = ci * ECHUNK
                    bo = buf * ECHUNK
                    pltpu.make_async_copy(
                        ed_hbm.at[pl.ds(off, ECHUNK)],
                        SB.at[pl.ds(bo, ECHUNK)], sem).start()
                    pltpu.make_async_copy(
                        w_hbm.at[pl.ds(off, ECHUNK)],
                        WB.at[pl.ds(bo, ECHUNK)], sem).start()

                def ewait(buf):
                    bo = buf * ECHUNK
                    pltpu.make_async_copy(
                        ed_hbm.at[pl.ds(0, ECHUNK)],
                        SB.at[pl.ds(bo, ECHUNK)], sem).wait()
                    pltpu.make_async_copy(
                        w_hbm.at[pl.ds(0, ECHUNK)],
                        WB.at[pl.ds(bo, ECHUNK)], sem).wait()

                def eprocess(buf):
                    bo = buf * ECHUNK

                    @plsc.parallel_loop(0, ECHUNK // 16, unroll=8)
                    def ebody(j):
                        pk = SB[pl.ds(bo + j * 16, 16)]
                        s = pk & 0xFFFF
                        d = lax.shift_right_logical(pk, 16)
                        wv = WB[pl.ds(bo + j * 16, 16)]
                        if half:
                            m = (d >= lo) & (d < lo + nh)
                            dl = jnp.where(m, d - lo, 0)
                        else:
                            dl = d
                        for c in range(k_ch):
                            v = plsc.load_gather(G, [s + c * n]) * wv
                            if half:
                                plsc.addupdate_scatter(
                                    A, [dl + c * nh], v, mask=m)
                            else:
                                plsc.addupdate_scatter(A, [dl + c * nh], v)

                nchunks = e // ECHUNK
                estart(0, 0)

                def epair(p, carry):
                    estart(2 * p + 1, 1)
                    ewait(0)
                    eprocess(0)

                    @pl.when(2 * p + 2 < nchunks)
                    def _():
                        estart(2 * p + 2, 0)
                    ewait(1)
                    eprocess(1)
                    return carry
                lax.fori_loop(0, nchunks // 2, epair, 0)

                for c in range(k_ch):
                    r = rbase + c
                    if step1:
                        pltpu.sync_copy(
                            A.at[pl.ds(c * nh, nh)],
                            out_hbm.at[pl.ds(r * n + lo, nh)])
                    else:
                        def wblk(bi, carry):
                            boff = bi * PCHUNK
                            pltpu.sync_copy(
                                prev_hbm.at[pl.ds(r * n + lo + boff, PCHUNK)],
                                PB)

                            def wb2(j, carry2):
                                v = (2.0 * A[pl.ds(c * nh + boff + j * 16, 16)]
                                     - PB[pl.ds(j * 16, 16)])
                                A[pl.ds(c * nh + boff + j * 16, 16)] = v
                                return carry2
                            lax.fori_loop(0, PCHUNK // 16, wb2, 0, unroll=8)
                            pltpu.sync_copy(
                                A.at[pl.ds(c * nh + boff, PCHUNK)],
                                out_hbm.at[pl.ds(r * n + lo + boff, PCHUNK)])
                            return carry
                        lax.fori_loop(0, nh // PCHUNK, wblk, 0)

    return pl.kernel(
        body,
        out_type=jax.ShapeDtypeStruct((R * n,), jnp.float32),
        mesh=_mesh(),
        compiler_params=pltpu.CompilerParams(needs_layout_passes=False),
        scratch_types=[
            pltpu.VMEM((k_ch * n,), jnp.float32),
            pltpu.VMEM((k_ch * nh,), jnp.float32),
            pltpu.VMEM((2 * ECHUNK,), jnp.int32),
            pltpu.VMEM((2 * ECHUNK,), jnp.float32),
            pltpu.VMEM((PCHUNK,), jnp.float32),
            pltpu.SemaphoreType.DMA,
        ],
    )


@functools.cache
def _make_partition(n):
    e = n * DEGREE
    se = e // NW
    stride = se
    mid = n // 2
    ec = min(ECHUNK, se)
    nck = se // ec

    def body(ed_hbm, w_hbm, ed2_hbm, w2_hbm, cnt_hbm,
             EB, WB, O0, W0, O1, W1, CB):
        wid = _worker_id()
        base = wid * se

        def zb(j, carry):
            zzf = jnp.zeros((16,), jnp.float32)
            zzi = jnp.zeros((16,), jnp.int32)
            W0[pl.ds(j * 16, 16)] = zzf
            W1[pl.ds(j * 16, 16)] = zzf
            O0[pl.ds(j * 16, 16)] = zzi
            O1[pl.ds(j * 16, 16)] = zzi
            return carry
        lax.fori_loop(0, se // 16, zb, 0, unroll=8)

        def chunk(ci, carry):
            pltpu.sync_copy(ed_hbm.at[pl.ds(base + ci * ec, ec)], EB)
            pltpu.sync_copy(w_hbm.at[pl.ds(base + ci * ec, ec)], WB)

            def ib(j, cc):
                c0, c1 = cc
                pk = EB[pl.ds(j * 16, 16)]
                wv = WB[pl.ds(j * 16, 16)]
                s = pk & 0xFFFF
                m0 = s < mid
                m1 = jnp.logical_not(m0)
                plsc.store_compressed(O0.at[pl.ds(c0, 16)], pk, mask=m0)
                plsc.store_compressed(W0.at[pl.ds(c0, 16)], wv, mask=m0)
                plsc.store_compressed(O1.at[pl.ds(c1, 16)], pk - mid, mask=m1)
                plsc.store_compressed(W1.at[pl.ds(c1, 16)], wv, mask=m1)
                n0 = jnp.sum(m0.astype(jnp.int32))
                return (c0 + n0, c1 + (16 - n0))
            return lax.fori_loop(0, ec // 16, ib, carry)
        c0, c1 = lax.fori_loop(0, nck, chunk,
                               (jnp.int32(0), jnp.int32(0)))

        io = lax.iota(jnp.int32, 16)
        CB[pl.ds(0, 16)] = jnp.where(io == 0, c0,
                                     jnp.where(io == 1, c1, 0))
        pltpu.sync_copy(CB, cnt_hbm.at[pl.ds(wid * 16, 16)])
        pltpu.sync_copy(O0, ed2_hbm.at[pl.ds((2 * wid) * stride, stride)])
        pltpu.sync_copy(W0, w2_hbm.at[pl.ds((2 * wid) * stride, stride)])
        pltpu.sync_copy(O1, ed2_hbm.at[pl.ds((2 * wid + 1) * stride, stride)])
        pltpu.sync_copy(W1, w2_hbm.at[pl.ds((2 * wid + 1) * stride, stride)])

    return pl.kernel(
        body,
        out_type=(jax.ShapeDtypeStruct((NW * 2 * stride,), jnp.int32),
                  jax.ShapeDtypeStruct((NW * 2 * stride,), jnp.float32),
                  jax.ShapeDtypeStruct((NW * 16,), jnp.int32)),
        mesh=_mesh(),
        compiler_params=pltpu.CompilerParams(needs_layout_passes=False),
        scratch_types=[
            pltpu.VMEM((ec,), jnp.int32),
            pltpu.VMEM((ec,), jnp.float32),
            pltpu.VMEM((stride,), jnp.int32),
            pltpu.VMEM((stride,), jnp.float32),
            pltpu.VMEM((stride,), jnp.int32),
            pltpu.VMEM((stride,), jnp.float32),
            pltpu.VMEM((16,), jnp.int32),
        ],
    )


K_CH_P = {65536: 1, 32768: 2, 16384: 4, 8192: 8, 4096: 16, 2048: 16}


@functools.cache
def _make_apply_part(R, n, step1):
    e = n * DEGREE
    nh = n // 2
    se = e // NW
    stride = se
    k_ch = K_CH_P[n]
    assert R % k_ch == 0
    groups = R // k_ch
    rounds = -(-groups // NW)
    ec = min(ECHUNK, stride)
    CPI = ec // 16

    def body(x_hbm, ed2_hbm, w2_hbm, cnt_hbm, prev_hbm, out_hbm,
             G, A, SB, WB, PB, CNTB, sem):
        wid = _worker_id()
        pltpu.sync_copy(cnt_hbm, CNTB)
        for rnd in range(rounds):
            gidx = rnd * NW + wid

            @pl.when(gidx < groups)
            def _():
                rbase = gidx * k_ch

                def zbody(j, carry):
                    zz = jnp.zeros((16,), jnp.float32)
                    for c in range(k_ch):
                        A[pl.ds(c * n + j * 16, 16)] = zz
                    return carry
                lax.fori_loop(0, n // 16, zbody, 0, unroll=8)

                for h in (0, 1):
                    for c in range(k_ch):
                        pltpu.sync_copy(
                            x_hbm.at[pl.ds((rbase + c) * n + h * nh, nh)],
                            G.at[pl.ds(c * nh, nh)])

                    def nit_of(t):
                        cnt = CNTB[pl.ds(t * 16, 16)][h]
                        return (cnt + 15) // 16

                    def cstart(rb, ci, par):
                        bo = (par & 1) * ec
                        pltpu.make_async_copy(
                            ed2_hbm.at[pl.ds(rb + ci * ec, ec)],
                            SB.at[pl.ds(bo, ec)], sem).start()
                        pltpu.make_async_copy(
                            w2_hbm.at[pl.ds(rb + ci * ec, ec)],
                            WB.at[pl.ds(bo, ec)], sem).start()

                    cstart(h * stride, 0, 0)

                    def region(t, gp):
                        nit = nit_of(t)
                        nck = jnp.maximum((nit + CPI - 1) // CPI, 1)
                        rb = (2 * t + h) * stride

                        def cloop(ci, gp2):
                            @pl.when(ci + 1 < nck)
                            def _():
                                cstart(rb, ci + 1, gp2 + 1)

                            @pl.when((ci + 1 >= nck) & (t + 1 < NW))
                            def _():
                                cstart((2 * (t + 1) + h) * stride, 0, gp2 + 1)
                            bo = (gp2 & 1) * ec
                            pltpu.make_async_copy(
                                ed2_hbm.at[pl.ds(0, ec)],
                                SB.at[pl.ds(bo, ec)], sem).wait()
                            pltpu.make_async_copy(
                                w2_hbm.at[pl.ds(0, ec)],
                                WB.at[pl.ds(bo, ec)], sem).wait()
                            jm = jnp.minimum(nit - ci * CPI, CPI)

                            @plsc.parallel_loop(0, jm, unroll=8)
                            def eb(j):
                                pk = SB[pl.ds(bo + j * 16, 16)]
                                s = pk & 0xFFFF
                                d = lax.shift_right_logical(pk, 16)
                                wv = WB[pl.ds(bo + j * 16, 16)]
                                for c in range(k_ch):
                                    v = plsc.load_gather(
                                        G, [s + c * nh]) * wv
                                    plsc.addupdate_scatter(
                                        A, [d + c * n], v)
                            return gp2 + 1
                        return lax.fori_loop(0, nck, cloop, gp)
                    lax.fori_loop(0, NW, region, jnp.int32(0))

                for c in range(k_ch):
                    r = rbase + c
                    if step1:
                        pltpu.sync_copy(A.at[pl.ds(c * n, n)],
                                        out_hbm.at[pl.ds(r * n, n)])
                    else:
                        def wblk(bi, carry):
                            boff = bi * PCHUNK
                            pltpu.sync_copy(
                                prev_hbm.at[pl.ds(r * n + boff, PCHUNK)], PB)

                            def wb2(j, carry2):
                                v = (2.0 * A[pl.ds(c * n + boff + j * 16, 16)]
                                     - PB[pl.ds(j * 16, 16)])
                                A[pl.ds(c * n + boff + j * 16, 16)] = v
                                return carry2
                            lax.fori_loop(0, PCHUNK // 16, wb2, 0, unroll=8)
                            pltpu.sync_copy(
                                A.at[pl.ds(c * n + boff, PCHUNK)],
                                out_hbm.at[pl.ds(r * n + boff, PCHUNK)])
                            return carry
                        lax.fori_loop(0, n // PCHUNK, wblk, 0)

    return pl.kernel(
        body,
        out_type=jax.ShapeDtypeStruct((R * n,), jnp.float32),
        mesh=_mesh(),
        compiler_params=pltpu.CompilerParams(needs_layout_passes=False),
        scratch_types=[
            pltpu.VMEM((k_ch * nh,), jnp.float32),
            pltpu.VMEM((k_ch * n,), jnp.float32),
            pltpu.VMEM((2 * ec,), jnp.int32),
            pltpu.VMEM((2 * ec,), jnp.float32),
            pltpu.VMEM((PCHUNK,), jnp.float32),
            pltpu.VMEM((NW * 16,), jnp.int32),
            pltpu.SemaphoreType.DMA,
        ],
    )


def _apply(xcm, g, prev):
    R, n = xcm.shape
    xf = xcm.reshape(R * n)
    pf = xf if prev is None else prev.reshape(R * n)
    if g[0] == 'p':
        _, ed2, w2, cnt = g
        out = _make_apply_part(R, n, prev is None)(xf, ed2, w2, cnt, pf)
    else:
        _, ed, w = g
        out = _make_apply(R, n, prev is None)(xf, ed, w, pf)
    return out.reshape(R, n)


@functools.cache
def _make_pool(R, n):
    nh = n // 2
    CB = min(2048, nh)
    rounds = -(-R // NW)

    def body(x_hbm, out_hbm, IB, OB):
        wid = _worker_id()
        iev = 2 * lax.iota(jnp.int32, 16)
        for rnd in range(rounds):
            r = rnd * NW + wid

            @pl.when(r < R)
            def _():
                def blk(bi, carry):
                    pltpu.sync_copy(
                        x_hbm.at[pl.ds(r * n + bi * 2 * CB, 2 * CB)], IB)

                    def ibody(j, carry2):
                        base = j * 32
                        a = plsc.load_gather(IB, [iev + base])
                        b = plsc.load_gather(IB, [iev + base + 1])
                        OB[pl.ds(j * 16, 16)] = jnp.maximum(a, b)
                        return carry2
                    lax.fori_loop(0, CB // 16, ibody, 0, unroll=8)
                    pltpu.sync_copy(
                        OB, out_hbm.at[pl.ds(r * nh + bi * CB, CB)])
                    return carry
                lax.fori_loop(0, nh // CB, blk, 0)

    return pl.kernel(
        body,
        out_type=jax.ShapeDtypeStruct((R * nh,), jnp.float32),
        mesh=_mesh(),
        compiler_params=pltpu.CompilerParams(needs_layout_passes=False),
        scratch_types=[
            pltpu.VMEM((2 * CB,), jnp.float32),
            pltpu.VMEM((CB,), jnp.float32),
        ],
    )


def _pool_cm(xcm):
    R, n = xcm.shape
    return _make_pool(R, n)(xcm.reshape(R * n)).reshape(R, n // 2)


@functools.cache
def _make_combine(Cin, Cout, n, mode, Cr=0):
    NB = 512
    grid = (B, n // NB)
    tx_spec = pl.BlockSpec((Cin, NB), lambda b, j: (b, j))
    w_spec = pl.BlockSpec((Cout, 4 * Cin), lambda b, j: (0, 0))
    b_spec = pl.BlockSpec((Cout, 1), lambda b, j: (0, 0))
    res_spec = pl.BlockSpec((Cr, NB), lambda b, j: (b, j)) if Cr else None
    in_specs = [tx_spec, tx_spec, tx_spec, tx_spec, w_spec, b_spec]
    if mode == 'res_w':
        in_specs += [pl.BlockSpec((Cout, Cr), lambda b, j: (0, 0)), res_spec]
    elif mode == 'res_id':
        in_specs += [res_spec]

    def body(t0, t1, t2, t3, wt, bias, *rest):
        out = rest[-1]
        a = jnp.concatenate([t0[...], t1[...], t2[...], t3[...]], axis=0)
        h = jnp.dot(wt[...], a, preferred_element_type=jnp.float32) + bias[...]
        if mode == 'res_w':
            h = h + jnp.dot(rest[0][...], rest[1][...],
                            preferred_element_type=jnp.float32)
        elif mode == 'res_id':
            h = h + rest[0][...]
        out[...] = jnp.maximum(h, 0.0)

    return pl.pallas_call(
        body,
        grid=grid,
        in_specs=in_specs,
        out_specs=pl.BlockSpec((Cout, NB), lambda b, j: (b, j)),
        out_shape=jax.ShapeDtypeStruct((B * Cout, n), jnp.float32),
    )


def _cheb_cm(xcm, g, W, bias, mode='plain', res=None, sW=None):
    R, n = xcm.shape
    Cin = R // B
    Cout = W.shape[2]
    tx0 = xcm
    tx1 = _apply(tx0, g, None)
    tx2 = _apply(tx1, g, tx0)
    tx3 = _apply(tx2, g, tx1)
    wt = W.transpose(2, 0, 1).reshape(Cout, K * Cin)
    bb = bias[:, None]
    args = [tx0, tx1, tx2, tx3, wt, bb]
    Cr = 0
    if mode == 'res_w':
        args += [sW.T, res]
        Cr = res.shape[0] // B
    elif mode == 'res_id':
        args += [res]
        Cr = res.shape[0] // B
    return _make_combine(Cin, Cout, n, mode, Cr)(*args)


def _block_cm(xcm, p, name, g):
    h = _cheb_cm(xcm, g, p[name + 'c1_W'], p[name + 'c1_b'])
    sW = p.get(name + 's_W')
    if sW is None:
        return _cheb_cm(h, g, p[name + 'c2_W'], p[name + 'c2_b'],
                        mode='res_id', res=xcm)
    return _cheb_cm(h, g, p[name + 'c2_W'], p[name + 'c2_b'],
                    mode='res_w', res=xcm, sW=sW)


def _to_bvc(xcm):
    R, n = xcm.shape
    return xcm.reshape(B, R // B, n).transpose(0, 2, 1)


def kernel(x, src5, dst5, w5, src4, dst4, w4, src3, dst3, w3, src2, dst2, w2, src1, dst1, w1, src0, dst0, w0, conv_W, conv_b, b5c1_W, b5c1_b, b5c2_W, b5c2_b, b5s_W, b4c1_W, b4c1_b, b4c2_W, b4c2_b, b4s_W, b3c1_W, b3c1_b, b3c2_W, b3c2_b, b3s_W, b2c1_W, b2c1_b, b2c2_W, b2c2_b, b2s_W, b1c1_W, b1c1_b, b1c2_W, b1c2_b, b0c1_W, b0c1_b, b0c2_W, b0c2_b):
    kw = dict(locals())
    graphs = {}
    for i, n_lvl in enumerate(LEVELS):
        lvl = 5 - i
        ed = kw['src%d' % lvl] | (kw['dst%d' % lvl] << 16)
        w_lvl = kw['w%d' % lvl]
        if n_lvl in K_CH_P:
            ed2, w2, cnt = _make_partition(n_lvl)(ed, w_lvl)
            graphs[lvl] = ('p', ed2, w2, cnt)
        else:
            graphs[lvl] = ('d', ed, w_lvl)
    p = {k: v for k, v in kw.items() if k.endswith('_W') or k.endswith('_b')}

    x3 = x.transpose(0, 2, 1)
    x3 = jnp.pad(x3, ((0, 0), (0, 2), (0, 0)))
    xcm = x3.reshape(B * 8, x.shape[1])
    conv_Wp = jnp.pad(p['conv_W'], ((0, 0), (0, 2), (0, 0)))
    h = _cheb_cm(xcm, graphs[5], conv_Wp, p['conv_b'])
    x5 = _block_cm(h, p, 'b5', graphs[5])
    x4 = _block_cm(_pool_cm(x5), p, 'b4', graphs[4])
    x3 = _block_cm(_pool_cm(x4), p, 'b3', graphs[3])
    x2 = _block_cm(_pool_cm(x3), p, 'b2', graphs[2])
    x1 = _block_cm(_pool_cm(x2), p, 'b1', graphs[1])
    x0 = _block_cm(_pool_cm(x1), p, 'b0', graphs[0])
    return tuple(_to_bvc(v) for v in (x0, x1, x2, x3, x4, x5))


# speedup vs baseline: 1.0531x; 1.0531x over previous
import functools

import jax
import jax.numpy as jnp
from jax import lax
from jax.experimental import pallas as pl
from jax.experimental.pallas import tpu as pltpu
from jax.experimental.pallas import tpu_sc as plsc

K = 4
B = 2
DEGREE = 8
LEVELS = [65536, 32768, 16384, 8192, 4096, 2048]
NC = 2
NS = 16
NW = NC * NS
ECHUNK = 2048
PCHUNK = 2048

K_CH = {65536: 1, 32768: 1, 16384: 2, 8192: 4, 4096: 8, 2048: 16}


def _mesh():
    return plsc.VectorSubcoreMesh(core_axis_name="c", subcore_axis_name="s",
                                  num_cores=NC, num_subcores=NS)


def _worker_id():
    return lax.axis_index("s") * NC + lax.axis_index("c")


@functools.cache
def _make_apply(R, n, step1):
    e = n * DEGREE
    half = n >= 65536
    H = 2 if half else 1
    nh = n // H
    k_ch = K_CH[n]
    assert R % k_ch == 0
    groups = R // k_ch
    items = groups * H
    rounds = -(-items // NW)

    def body(x_hbm, ed_hbm, w_hbm, prev_hbm, out_hbm,
             G, A, SB, WB, PB, sem):
        wid = _worker_id()
        for rnd in range(rounds):
            item = rnd * NW + wid

            @pl.when(item < items)
            def _():
                g = item // H
                h = item % H
                lo = h * nh
                rbase = g * k_ch
                for c in range(k_ch):
                    pltpu.sync_copy(x_hbm.at[pl.ds((rbase + c) * n, n)],
                                    G.at[pl.ds(c * n, n)])

                def zbody(j, carry):
                    zz = jnp.zeros((16,), jnp.float32)
                    for c in range(k_ch):
                        A[pl.ds(c * nh + j * 16, 16)] = zz
                    return carry
                lax.fori_loop(0, nh // 16, zbody, 0, unroll=8)

                def estart(ci, buf):
                    off = ci * ECHUNK
                    bo = buf * ECHUNK
                    pltpu.make_async_copy(
                        ed_hbm.at[pl.ds(off, ECHUNK)],
                        SB.at[pl.ds(bo, ECHUNK)], sem).start()
                    pltpu.make_async_copy(
                        w_hbm.at[pl.ds(off, ECHUNK)],
                        WB.at[pl.ds(bo, ECHUNK)], sem).start()

                def ewait(buf):
                    bo = buf * ECHUNK
                    pltpu.make_async_copy(
                        ed_hbm.at[pl.ds(0, ECHUNK)],
                        SB.at[pl.ds(bo, ECHUNK)], sem).wait()
                    pltpu.make_async_copy(
                        w_hbm.at[pl.ds(0, ECHUNK)],
                        WB.at[pl.ds(bo, ECHUNK)], sem).wait()

                def eprocess(buf):
                    bo = buf * ECHUNK

                    @plsc.parallel_loop(0, ECHUNK // 16, unroll=8)
                    def ebody(j):
                        pk = SB[pl.ds(bo + j * 16, 16)]
                        s = pk & 0xFFFF
                        d = lax.shift_right_logical(pk, 16)
                        wv = WB[pl.ds(bo + j * 16, 16)]
                        if half:
                            m = (d >= lo) & (d < lo + nh)
                            dl = jnp.where(m, d - lo, 0)
                        else:
                            dl = d
                        for c in range(k_ch):
                            v = plsc.load_gather(G, [s + c * n]) * wv
                            if half:
                                plsc.addupdate_scatter(
                                    A, [dl + c * nh], v, mask=m)
                            else:
                                plsc.addupdate_scatter(A, [dl + c * nh], v)

                nchunks = e // ECHUNK
                estart(0, 0)

                def epair(p, carry):
                    estart(2 * p + 1, 1)
                    ewait(0)
                    eprocess(0)

                    @pl.when(2 * p + 2 < nchunks)
                    def _():
                        estart(2 * p + 2, 0)
                    ewait(1)
                    eprocess(1)
                    return carry
                lax.fori_loop(0, nchunks // 2, epair, 0)

                for c in range(k_ch):
                    r = rbase + c
                    if step1:
                        pltpu.sync_copy(
                            A.at[pl.ds(c * nh, nh)],
                            out_hbm.at[pl.ds(r * n + lo, nh)])
                    else:
                        def wblk(bi, carry):
                            boff = bi * PCHUNK
                            pltpu.sync_copy(
                                prev_hbm.at[pl.ds(r * n + lo + boff, PCHUNK)],
                                PB)

                            def wb2(j, carry2):
                                v = (2.0 * A[pl.ds(c * nh + boff + j * 16, 16)]
                                     - PB[pl.ds(j * 16, 16)])
                                A[pl.ds(c * nh + boff + j * 16, 16)] = v
                                return carry2
                            lax.fori_loop(0, PCHUNK // 16, wb2, 0, unroll=8)
                            pltpu.sync_copy(
                                A.at[pl.ds(c * nh + boff, PCHUNK)],
                                out_hbm.at[pl.ds(r * n + lo + boff, PCHUNK)])
                            return carry
                        lax.fori_loop(0, nh // PCHUNK, wblk, 0)

    return pl.kernel(
        body,
        out_type=jax.ShapeDtypeStruct((R * n,), jnp.float32),
        mesh=_mesh(),
        compiler_params=pltpu.CompilerParams(needs_layout_passes=False),
        scratch_types=[
            pltpu.VMEM((k_ch * n,), jnp.float32),
            pltpu.VMEM((k_ch * nh,), jnp.float32),
            pltpu.VMEM((2 * ECHUNK,), jnp.int32),
            pltpu.VMEM((2 * ECHUNK,), jnp.float32),
            pltpu.VMEM((PCHUNK,), jnp.float32),
            pltpu.SemaphoreType.DMA,
        ],
    )


@functools.cache
def _make_partition(n):
    e = n * DEGREE
    se = e // NW
    stride = se
    mid = n // 2
    ec = min(ECHUNK, se)
    nck = se // ec

    def body(ed_hbm, w_hbm, ed2_hbm, w2_hbm, cnt_hbm,
             EB, WB, O0, W0, O1, W1, CB):
        wid = _worker_id()
        base = wid * se

        def zb(j, carry):
            zzf = jnp.zeros((16,), jnp.float32)
            zzi = jnp.zeros((16,), jnp.int32)
            W0[pl.ds(j * 16, 16)] = zzf
            W1[pl.ds(j * 16, 16)] = zzf
            O0[pl.ds(j * 16, 16)] = zzi
            O1[pl.ds(j * 16, 16)] = zzi
            return carry
        lax.fori_loop(0, se // 16, zb, 0, unroll=8)

        def chunk(ci, carry):
            pltpu.sync_copy(ed_hbm.at[pl.ds(base + ci * ec, ec)], EB)
            pltpu.sync_copy(w_hbm.at[pl.ds(base + ci * ec, ec)], WB)

            def ib(j, cc):
                c0, c1 = cc
                pk = EB[pl.ds(j * 16, 16)]
                wv = WB[pl.ds(j * 16, 16)]
                s = pk & 0xFFFF
                m0 = s < mid
                m1 = jnp.logical_not(m0)
                plsc.store_compressed(O0.at[pl.ds(c0, 16)], pk, mask=m0)
                plsc.store_compressed(W0.at[pl.ds(c0, 16)], wv, mask=m0)
                plsc.store_compressed(O1.at[pl.ds(c1, 16)], pk - mid, mask=m1)
                plsc.store_compressed(W1.at[pl.ds(c1, 16)], wv, mask=m1)
                n0 = jnp.sum(m0.astype(jnp.int32))
                return (c0 + n0, c1 + (16 - n0))
            return lax.fori_loop(0, ec // 16, ib, carry)
        c0, c1 = lax.fori_loop(0, nck, chunk,
                               (jnp.int32(0), jnp.int32(0)))

        io = lax.iota(jnp.int32, 16)
        CB[pl.ds(0, 16)] = jnp.where(io == 0, c0,
                                     jnp.where(io == 1, c1, 0))
        pltpu.sync_copy(CB, cnt_hbm.at[pl.ds(wid * 16, 16)])
        pltpu.sync_copy(O0, ed2_hbm.at[pl.ds((2 * wid) * stride, stride)])
        pltpu.sync_copy(W0, w2_hbm.at[pl.ds((2 * wid) * stride, stride)])
        pltpu.sync_copy(O1, ed2_hbm.at[pl.ds((2 * wid + 1) * stride, stride)])
        pltpu.sync_copy(W1, w2_hbm.at[pl.ds((2 * wid + 1) * stride, stride)])

    return pl.kernel(
        body,
        out_type=(jax.ShapeDtypeStruct((NW * 2 * stride,), jnp.int32),
                  jax.ShapeDtypeStruct((NW * 2 * stride,), jnp.float32),
                  jax.ShapeDtypeStruct((NW * 16,), jnp.int32)),
        mesh=_mesh(),
        compiler_params=pltpu.CompilerParams(needs_layout_passes=False),
        scratch_types=[
            pltpu.VMEM((ec,), jnp.int32),
            pltpu.VMEM((ec,), jnp.float32),
            pltpu.VMEM((stride,), jnp.int32),
            pltpu.VMEM((stride,), jnp.float32),
            pltpu.VMEM((stride,), jnp.int32),
            pltpu.VMEM((stride,), jnp.float32),
            pltpu.VMEM((16,), jnp.int32),
        ],
    )


K_CH_P = {65536: 1, 32768: 2, 16384: 4}


@functools.cache
def _make_apply_part(R, n, step1):
    e = n * DEGREE
    nh = n // 2
    se = e // NW
    stride = se
    k_ch = K_CH_P[n]
    assert R % k_ch == 0
    groups = R // k_ch
    rounds = -(-groups // NW)
    ec = min(ECHUNK, stride)
    CPI = ec // 16

    def body(x_hbm, ed2_hbm, w2_hbm, cnt_hbm, prev_hbm, out_hbm,
             G, A, SB, WB, PB, CNTB, sem):
        wid = _worker_id()
        pltpu.sync_copy(cnt_hbm, CNTB)
        for rnd in range(rounds):
            gidx = rnd * NW + wid

            @pl.when(gidx < groups)
            def _():
                rbase = gidx * k_ch

                def zbody(j, carry):
                    zz = jnp.zeros((16,), jnp.float32)
                    for c in range(k_ch):
                        A[pl.ds(c * n + j * 16, 16)] = zz
                    return carry
                lax.fori_loop(0, n // 16, zbody, 0, unroll=8)

                for h in (0, 1):
                    for c in range(k_ch):
                        pltpu.sync_copy(
                            x_hbm.at[pl.ds((rbase + c) * n + h * nh, nh)],
                            G.at[pl.ds(c * nh, nh)])

                    def nit_of(t):
                        cnt = CNTB[pl.ds(t * 16, 16)][h]
                        return (cnt + 15) // 16

                    def cstart(rb, ci, par):
                        bo = (par & 1) * ec
                        pltpu.make_async_copy(
                            ed2_hbm.at[pl.ds(rb + ci * ec, ec)],
                            SB.at[pl.ds(bo, ec)], sem).start()
                        pltpu.make_async_copy(
                            w2_hbm.at[pl.ds(rb + ci * ec, ec)],
                            WB.at[pl.ds(bo, ec)], sem).start()

                    cstart(h * stride, 0, 0)

                    def region(t, gp):
                        nit = nit_of(t)
                        nck = jnp.maximum((nit + CPI - 1) // CPI, 1)
                        rb = (2 * t + h) * stride

                        def cloop(ci, gp2):
                            @pl.when(ci + 1 < nck)
                            def _():
                                cstart(rb, ci + 1, gp2 + 1)

                            @pl.when((ci + 1 >= nck) & (t + 1 < NW))
                            def _():
                                cstart((2 * (t + 1) + h) * stride, 0, gp2 + 1)
                            bo = (gp2 & 1) * ec
                            pltpu.make_async_copy(
                                ed2_hbm.at[pl.ds(0, ec)],
                                SB.at[pl.ds(bo, ec)], sem).wait()
                            pltpu.make_async_copy(
                                w2_hbm.at[pl.ds(0, ec)],
                                WB.at[pl.ds(bo, ec)], sem).wait()
                            jm = jnp.minimum(nit - ci * CPI, CPI)

                            @plsc.parallel_loop(0, jm, unroll=8)
                            def eb(j):
                                pk = SB[pl.ds(bo + j * 16, 16)]
                                s = pk & 0xFFFF
                                d = lax.shift_right_logical(pk, 16)
                                wv = WB[pl.ds(bo + j * 16, 16)]
                                for c in range(k_ch):
                                    v = plsc.load_gather(
                                        G, [s + c * nh]) * wv
                                    plsc.addupdate_scatter(
                                        A, [d + c * n], v)
                            return gp2 + 1
                        return lax.fori_loop(0, nck, cloop, gp)
                    lax.fori_loop(0, NW, region, jnp.int32(0))

                for c in range(k_ch):
                    r = rbase + c
                    if step1:
                        pltpu.sync_copy(A.at[pl.ds(c * n, n)],
                                        out_hbm.at[pl.ds(r * n, n)])
                    else:
                        def wblk(bi, carry):
                            boff = bi * PCHUNK
                            pltpu.sync_copy(
                                prev_hbm.at[pl.ds(r * n + boff, PCHUNK)], PB)

                            def wb2(j, carry2):
                                v = (2.0 * A[pl.ds(c * n + boff + j * 16, 16)]
                                     - PB[pl.ds(j * 16, 16)])
                                A[pl.ds(c * n + boff + j * 16, 16)] = v
                                return carry2
                            lax.fori_loop(0, PCHUNK // 16, wb2, 0, unroll=8)
                            pltpu.sync_copy(
                                A.at[pl.ds(c * n + boff, PCHUNK)],
                                out_hbm.at[pl.ds(r * n + boff, PCHUNK)])
                            return carry
                        lax.fori_loop(0, n // PCHUNK, wblk, 0)

    return pl.kernel(
        body,
        out_type=jax.ShapeDtypeStruct((R * n,), jnp.float32),
        mesh=_mesh(),
        compiler_params=pltpu.CompilerParams(needs_layout_passes=False),
        scratch_types=[
            pltpu.VMEM((k_ch * nh,), jnp.float32),
            pltpu.VMEM((k_ch * n,), jnp.float32),
            pltpu.VMEM((2 * ec,), jnp.int32),
            pltpu.VMEM((2 * ec,), jnp.float32),
            pltpu.VMEM((PCHUNK,), jnp.float32),
            pltpu.VMEM((NW * 16,), jnp.int32),
            pltpu.SemaphoreType.DMA,
        ],
    )


def _apply(xcm, g, prev):
    R, n = xcm.shape
    xf = xcm.reshape(R * n)
    pf = xf if prev is None else prev.reshape(R * n)
    if g[0] == 'p':
        _, ed2, w2, cnt = g
        out = _make_apply_part(R, n, prev is None)(xf, ed2, w2, cnt, pf)
    else:
        _, ed, w = g
        out = _make_apply(R, n, prev is None)(xf, ed, w, pf)
    return out.reshape(R, n)


@functools.cache
def _make_pool(R, n):
    nh = n // 2
    CB = min(2048, nh)
    rounds = -(-R // NW)

    def body(x_hbm, out_hbm, IB, OB):
        wid = _worker_id()
        iev = 2 * lax.iota(jnp.int32, 16)
        for rnd in range(rounds):
            r = rnd * NW + wid

            @pl.when(r < R)
            def _():
                def blk(bi, carry):
                    pltpu.sync_copy(
                        x_hbm.at[pl.ds(r * n + bi * 2 * CB, 2 * CB)], IB)

                    def ibody(j, carry2):
                        base = j * 32
                        a = plsc.load_gather(IB, [iev + base])
                        b = plsc.load_gather(IB, [iev + base + 1])
                        OB[pl.ds(j * 16, 16)] = jnp.maximum(a, b)
                        return carry2
                    lax.fori_loop(0, CB // 16, ibody, 0, unroll=8)
                    pltpu.sync_copy(
                        OB, out_hbm.at[pl.ds(r * nh + bi * CB, CB)])
                    return carry
                lax.fori_loop(0, nh // CB, blk, 0)

    return pl.kernel(
        body,
        out_type=jax.ShapeDtypeStruct((R * nh,), jnp.float32),
        mesh=_mesh(),
        compiler_params=pltpu.CompilerParams(needs_layout_passes=False),
        scratch_types=[
            pltpu.VMEM((2 * CB,), jnp.float32),
            pltpu.VMEM((CB,), jnp.float32),
        ],
    )


def _pool_cm(xcm):
    R, n = xcm.shape
    return _make_pool(R, n)(xcm.reshape(R * n)).reshape(R, n // 2)


@functools.cache
def _make_combine(Cin, Cout, n, mode, Cr=0):
    NB = 512
    grid = (B, n // NB)
    tx_spec = pl.BlockSpec((Cin, NB), lambda b, j: (b, j))
    w_spec = pl.BlockSpec((Cout, 4 * Cin), lambda b, j: (0, 0))
    b_spec = pl.BlockSpec((Cout, 1), lambda b, j: (0, 0))
    res_spec = pl.BlockSpec((Cr, NB), lambda b, j: (b, j)) if Cr else None
    in_specs = [tx_spec, tx_spec, tx_spec, tx_spec, w_spec, b_spec]
    if mode == 'res_w':
        in_specs += [pl.BlockSpec((Cout, Cr), lambda b, j: (0, 0)), res_spec]
    elif mode == 'res_id':
        in_specs += [res_spec]

    def body(t0, t1, t2, t3, wt, bias, *rest):
        out = rest[-1]
        a = jnp.concatenate([t0[...], t1[...], t2[...], t3[...]], axis=0)
        h = jnp.dot(wt[...], a, preferred_element_type=jnp.float32) + bias[...]
        if mode == 'res_w':
            h = h + jnp.dot(rest[0][...], rest[1][...],
                            preferred_element_type=jnp.float32)
        elif mode == 'res_id':
            h = h + rest[0][...]
        out[...] = jnp.maximum(h, 0.0)

    return pl.pallas_call(
        body,
        grid=grid,
        in_specs=in_specs,
        out_specs=pl.BlockSpec((Cout, NB), lambda b, j: (b, j)),
        out_shape=jax.ShapeDtypeStruct((B * Cout, n), jnp.float32),
    )


def _cheb_cm(xcm, g, W, bias, mode='plain', res=None, sW=None):
    R, n = xcm.shape
    Cin = R // B
    Cout = W.shape[2]
    tx0 = xcm
    tx1 = _apply(tx0, g, None)
    tx2 = _apply(tx1, g, tx0)
    tx3 = _apply(tx2, g, tx1)
    wt = W.transpose(2, 0, 1).reshape(Cout, K * Cin)
    bb = bias[:, None]
    args = [tx0, tx1, tx2, tx3, wt, bb]
    Cr = 0
    if mode == 'res_w':
        args += [sW.T, res]
        Cr = res.shape[0] // B
    elif mode == 'res_id':
        args += [res]
        Cr = res.shape[0] // B
    return _make_combine(Cin, Cout, n, mode, Cr)(*args)


def _block_cm(xcm, p, name, g):
    h = _cheb_cm(xcm, g, p[name + 'c1_W'], p[name + 'c1_b'])
    sW = p.get(name + 's_W')
    if sW is None:
        return _cheb_cm(h, g, p[name + 'c2_W'], p[name + 'c2_b'],
                        mode='res_id', res=xcm)
    return _cheb_cm(h, g, p[name + 'c2_W'], p[name + 'c2_b'],
                    mode='res_w', res=xcm, sW=sW)


def _to_bvc(xcm):
    R, n = xcm.shape
    return xcm.reshape(B, R // B, n).transpose(0, 2, 1)


def kernel(x, src5, dst5, w5, src4, dst4, w4, src3, dst3, w3, src2, dst2, w2, src1, dst1, w1, src0, dst0, w0, conv_W, conv_b, b5c1_W, b5c1_b, b5c2_W, b5c2_b, b5s_W, b4c1_W, b4c1_b, b4c2_W, b4c2_b, b4s_W, b3c1_W, b3c1_b, b3c2_W, b3c2_b, b3s_W, b2c1_W, b2c1_b, b2c2_W, b2c2_b, b2s_W, b1c1_W, b1c1_b, b1c2_W, b1c2_b, b0c1_W, b0c1_b, b0c2_W, b0c2_b):
    kw = dict(locals())
    graphs = {}
    for i, n_lvl in enumerate(LEVELS):
        lvl = 5 - i
        ed = kw['src%d' % lvl] | (kw['dst%d' % lvl] << 16)
        w_lvl = kw['w%d' % lvl]
        if n_lvl in K_CH_P:
            ed2, w2, cnt = _make_partition(n_lvl)(ed, w_lvl)
            graphs[lvl] = ('p', ed2, w2, cnt)
        else:
            graphs[lvl] = ('d', ed, w_lvl)
    p = {k: v for k, v in kw.items() if k.endswith('_W') or k.endswith('_b')}

    x3 = x.transpose(0, 2, 1)
    x3 = jnp.pad(x3, ((0, 0), (0, 2), (0, 0)))
    xcm = x3.reshape(B * 8, x.shape[1])
    conv_Wp = jnp.pad(p['conv_W'], ((0, 0), (0, 2), (0, 0)))
    h = _cheb_cm(xcm, graphs[5], conv_Wp, p['conv_b'])
    x5 = _block_cm(h, p, 'b5', graphs[5])
    x4 = _block_cm(_pool_cm(x5), p, 'b4', graphs[4])
    x3 = _block_cm(_pool_cm(x4), p, 'b3', graphs[3])
    x2 = _block_cm(_pool_cm(x3), p, 'b2', graphs[2])
    x1 = _block_cm(_pool_cm(x2), p, 'b1', graphs[1])
    x0 = _block_cm(_pool_cm(x1), p, 'b0', graphs[0])
    return tuple(_to_bvc(v) for v in (x0, x1, x2, x3, x4, x5))


# speedup vs baseline: 1.0686x; 1.0147x over previous
import functools

import jax
import jax.numpy as jnp
from jax import lax
from jax.experimental import pallas as pl
from jax.experimental.pallas import tpu as pltpu
from jax.experimental.pallas import tpu_sc as plsc

K = 4
B = 2
DEGREE = 8
LEVELS = [65536, 32768, 16384, 8192, 4096, 2048]
NC = 2
NS = 16
NW = NC * NS
ECHUNK = 2048
PCHUNK = 2048

K_CH = {65536: 1, 32768: 1, 16384: 2, 8192: 4, 4096: 8, 2048: 16}


def _mesh():
    return plsc.VectorSubcoreMesh(core_axis_name="c", subcore_axis_name="s",
                                  num_cores=NC, num_subcores=NS)


def _worker_id():
    return lax.axis_index("s") * NC + lax.axis_index("c")


@functools.cache
def _make_apply(R, n, step1):
    e = n * DEGREE
    half = n >= 65536
    H = 2 if half else 1
    nh = n // H
    k_ch = K_CH[n]
    assert R % k_ch == 0
    groups = R // k_ch
    items = groups * H
    rounds = -(-items // NW)

    def body(x_hbm, ed_hbm, w_hbm, prev_hbm, out_hbm,
             G, A, SB, WB, PB, sem):
        wid = _worker_id()
        for rnd in range(rounds):
            item = rnd * NW + wid

            @pl.when(item < items)
            def _():
                g = item // H
                h = item % H
                lo = h * nh
                rbase = g * k_ch
                for c in range(k_ch):
                    pltpu.sync_copy(x_hbm.at[pl.ds((rbase + c) * n, n)],
                                    G.at[pl.ds(c * n, n)])

                def zbody(j, carry):
                    zz = jnp.zeros((16,), jnp.float32)
                    for c in range(k_ch):
                        A[pl.ds(c * nh + j * 16, 16)] = zz
                    return carry
                lax.fori_loop(0, nh // 16, zbody, 0, unroll=8)

                def estart(ci, buf):
                    off = ci * ECHUNK
                    bo = buf * ECHUNK
                    pltpu.make_async_copy(
                        ed_hbm.at[pl.ds(off, ECHUNK)],
                        SB.at[pl.ds(bo, ECHUNK)], sem).start()
                    pltpu.make_async_copy(
                        w_hbm.at[pl.ds(off, ECHUNK)],
                        WB.at[pl.ds(bo, ECHUNK)], sem).start()

                def ewait(buf):
                    bo = buf * ECHUNK
                    pltpu.make_async_copy(
                        ed_hbm.at[pl.ds(0, ECHUNK)],
                        SB.at[pl.ds(bo, ECHUNK)], sem).wait()
                    pltpu.make_async_copy(
                        w_hbm.at[pl.ds(0, ECHUNK)],
                        WB.at[pl.ds(bo, ECHUNK)], sem).wait()

                def eprocess(buf):
                    bo = buf * ECHUNK

                    @plsc.parallel_loop(0, ECHUNK // 16, unroll=8)
                    def ebody(j):
                        pk = SB[pl.ds(bo + j * 16, 16)]
                        s = pk & 0xFFFF
                        d = lax.shift_right_logical(pk, 16)
                        wv = WB[pl.ds(bo + j * 16, 16)]
                        if half:
                            m = (d >= lo) & (d < lo + nh)
                            dl = jnp.where(m, d - lo, 0)
                        else:
                            dl = d
                        for c in range(k_ch):
                            v = plsc.load_gather(G, [s + c * n]) * wv
                            if half:
                                plsc.addupdate_scatter(
                                    A, [dl + c * nh], v, mask=m)
                            else:
                                plsc.addupdate_scatter(A, [dl + c * nh], v)

                nchunks = e // ECHUNK
                estart(0, 0)

                def epair(p, carry):
                    estart(2 * p + 1, 1)
                    ewait(0)
                    eprocess(0)

                    @pl.when(2 * p + 2 < nchunks)
                    def _():
                        estart(2 * p + 2, 0)
                    ewait(1)
                    eprocess(1)
                    return carry
                lax.fori_loop(0, nchunks // 2, epair, 0)

                for c in range(k_ch):
                    r = rbase + c
                    if step1:
                        pltpu.sync_copy(
                            A.at[pl.ds(c * nh, nh)],
                            out_hbm.at[pl.ds(r * n + lo, nh)])
                    else:
                        def wblk(bi, carry):
                            boff = bi * PCHUNK
                            pltpu.sync_copy(
                                prev_hbm.at[pl.ds(r * n + lo + boff, PCHUNK)],
                                PB)

                            def wb2(j, carry2):
                                v = (2.0 * A[pl.ds(c * nh + boff + j * 16, 16)]
                                     - PB[pl.ds(j * 16, 16)])
                                A[pl.ds(c * nh + boff + j * 16, 16)] = v
                                return carry2
                            lax.fori_loop(0, PCHUNK // 16, wb2, 0, unroll=8)
                            pltpu.sync_copy(
                                A.at[pl.ds(c * nh + boff, PCHUNK)],
                                out_hbm.at[pl.ds(r * n + lo + boff, PCHUNK)])
                            return carry
                        lax.fori_loop(0, nh // PCHUNK, wblk, 0)

    return pl.kernel(
        body,
        out_type=jax.ShapeDtypeStruct((R * n,), jnp.float32),
        mesh=_mesh(),
        compiler_params=pltpu.CompilerParams(needs_layout_passes=False),
        scratch_types=[
            pltpu.VMEM((k_ch * n,), jnp.float32),
            pltpu.VMEM((k_ch * nh,), jnp.float32),
            pltpu.VMEM((2 * ECHUNK,), jnp.int32),
            pltpu.VMEM((2 * ECHUNK,), jnp.float32),
            pltpu.VMEM((PCHUNK,), jnp.float32),
            pltpu.SemaphoreType.DMA,
        ],
    )


@functools.cache
def _make_partition(n):
    e = n * DEGREE
    se = e // NW
    stride = se
    mid = n // 2
    ec = min(ECHUNK, se)
    nck = se // ec

    def body(ed_hbm, w_hbm, ed2_hbm, w2_hbm, cnt_hbm,
             EB, WB, O0, W0, O1, W1, CB):
        wid = _worker_id()
        base = wid * se

        def zb(j, carry):
            zzf = jnp.zeros((16,), jnp.float32)
            zzi = jnp.zeros((16,), jnp.int32)
            W0[pl.ds(j * 16, 16)] = zzf
            W1[pl.ds(j * 16, 16)] = zzf
            O0[pl.ds(j * 16, 16)] = zzi
            O1[pl.ds(j * 16, 16)] = zzi
            return carry
        lax.fori_loop(0, se // 16, zb, 0, unroll=8)

        def chunk(ci, carry):
            pltpu.sync_copy(ed_hbm.at[pl.ds(base + ci * ec, ec)], EB)
            pltpu.sync_copy(w_hbm.at[pl.ds(base + ci * ec, ec)], WB)

            def ib(j, cc):
                c0, c1 = cc
                pk = EB[pl.ds(j * 16, 16)]
                wv = WB[pl.ds(j * 16, 16)]
                s = pk & 0xFFFF
                m0 = s < mid
                m1 = jnp.logical_not(m0)
                plsc.store_compressed(O0.at[pl.ds(c0, 16)], pk, mask=m0)
                plsc.store_compressed(W0.at[pl.ds(c0, 16)], wv, mask=m0)
                plsc.store_compressed(O1.at[pl.ds(c1, 16)], pk - mid, mask=m1)
                plsc.store_compressed(W1.at[pl.ds(c1, 16)], wv, mask=m1)
                n0 = jnp.sum(m0.astype(jnp.int32))
                return (c0 + n0, c1 + (16 - n0))
            return lax.fori_loop(0, ec // 16, ib, carry)
        c0, c1 = lax.fori_loop(0, nck, chunk,
                               (jnp.int32(0), jnp.int32(0)))

        io = lax.iota(jnp.int32, 16)
        CB[pl.ds(0, 16)] = jnp.where(io == 0, c0,
                                     jnp.where(io == 1, c1, 0))
        pltpu.sync_copy(CB, cnt_hbm.at[pl.ds(wid * 16, 16)])
        pltpu.sync_copy(O0, ed2_hbm.at[pl.ds((2 * wid) * stride, stride)])
        pltpu.sync_copy(W0, w2_hbm.at[pl.ds((2 * wid) * stride, stride)])
        pltpu.sync_copy(O1, ed2_hbm.at[pl.ds((2 * wid + 1) * stride, stride)])
        pltpu.sync_copy(W1, w2_hbm.at[pl.ds((2 * wid + 1) * stride, stride)])

    return pl.kernel(
        body,
        out_type=(jax.ShapeDtypeStruct((NW * 2 * stride,), jnp.int32),
                  jax.ShapeDtypeStruct((NW * 2 * stride,), jnp.float32),
                  jax.ShapeDtypeStruct((NW * 16,), jnp.int32)),
        mesh=_mesh(),
        compiler_params=pltpu.CompilerParams(needs_layout_passes=False),
        scratch_types=[
            pltpu.VMEM((ec,), jnp.int32),
            pltpu.VMEM((ec,), jnp.float32),
            pltpu.VMEM((stride,), jnp.int32),
            pltpu.VMEM((stride,), jnp.float32),
            pltpu.VMEM((stride,), jnp.int32),
            pltpu.VMEM((stride,), jnp.float32),
            pltpu.VMEM((16,), jnp.int32),
        ],
    )


K_CH_P = {65536: 1, 32768: 2}


@functools.cache
def _make_apply_part(R, n, step1):
    e = n * DEGREE
    nh = n // 2
    se = e // NW
    stride = se
    k_ch = K_CH_P[n]
    assert R % k_ch == 0
    groups = R // k_ch
    rounds = -(-groups // NW)
    ec = min(ECHUNK, stride)
    CPI = ec // 16

    def body(x_hbm, ed2_hbm, w2_hbm, cnt_hbm, prev_hbm, out_hbm,
             G, A, SB, WB, PB, CNTB, sem):
        wid = _worker_id()
        pltpu.sync_copy(cnt_hbm, CNTB)
        for rnd in range(rounds):
            gidx = rnd * NW + wid

            @pl.when(gidx < groups)
            def _():
                rbase = gidx * k_ch

                def zbody(j, carry):
                    zz = jnp.zeros((16,), jnp.float32)
                    for c in range(k_ch):
                        A[pl.ds(c * n + j * 16, 16)] = zz
                    return carry
                lax.fori_loop(0, n // 16, zbody, 0, unroll=8)

                for h in (0, 1):
                    for c in range(k_ch):
                        pltpu.sync_copy(
                            x_hbm.at[pl.ds((rbase + c) * n + h * nh, nh)],
                            G.at[pl.ds(c * nh, nh)])

                    def nit_of(t):
                        cnt = CNTB[pl.ds(t * 16, 16)][h]
                        return (cnt + 15) // 16

                    def cstart(rb, ci, par):
                        bo = (par & 1) * ec
                        pltpu.make_async_copy(
                            ed2_hbm.at[pl.ds(rb + ci * ec, ec)],
                            SB.at[pl.ds(bo, ec)], sem).start()
                        pltpu.make_async_copy(
                            w2_hbm.at[pl.ds(rb + ci * ec, ec)],
                            WB.at[pl.ds(bo, ec)], sem).start()

                    cstart(h * stride, 0, 0)

                    def region(t, gp):
                        nit = nit_of(t)
                        nck = jnp.maximum((nit + CPI - 1) // CPI, 1)
                        rb = (2 * t + h) * stride

                        def cloop(ci, gp2):
                            @pl.when(ci + 1 < nck)
                            def _():
                                cstart(rb, ci + 1, gp2 + 1)

                            @pl.when((ci + 1 >= nck) & (t + 1 < NW))
                            def _():
                                cstart((2 * (t + 1) + h) * stride, 0, gp2 + 1)
                            bo = (gp2 & 1) * ec
                            pltpu.make_async_copy(
                                ed2_hbm.at[pl.ds(0, ec)],
                                SB.at[pl.ds(bo, ec)], sem).wait()
                            pltpu.make_async_copy(
                                w2_hbm.at[pl.ds(0, ec)],
                                WB.at[pl.ds(bo, ec)], sem).wait()
                            jm = jnp.minimum(nit - ci * CPI, CPI)

                            @plsc.parallel_loop(0, jm, unroll=8)
                            def eb(j):
                                pk = SB[pl.ds(bo + j * 16, 16)]
                                s = pk & 0xFFFF
                                d = lax.shift_right_logical(pk, 16)
                                wv = WB[pl.ds(bo + j * 16, 16)]
                                for c in range(k_ch):
                                    v = plsc.load_gather(
                                        G, [s + c * nh]) * wv
                                    plsc.addupdate_scatter(
                                        A, [d + c * n], v)
                            return gp2 + 1
                        return lax.fori_loop(0, nck, cloop, gp)
                    lax.fori_loop(0, NW, region, jnp.int32(0))

                for c in range(k_ch):
                    r = rbase + c
                    if step1:
                        pltpu.sync_copy(A.at[pl.ds(c * n, n)],
                                        out_hbm.at[pl.ds(r * n, n)])
                    else:
                        def wblk(bi, carry):
                            boff = bi * PCHUNK
                            pltpu.sync_copy(
                                prev_hbm.at[pl.ds(r * n + boff, PCHUNK)], PB)

                            def wb2(j, carry2):
                                v = (2.0 * A[pl.ds(c * n + boff + j * 16, 16)]
                                     - PB[pl.ds(j * 16, 16)])
                                A[pl.ds(c * n + boff + j * 16, 16)] = v
                                return carry2
                            lax.fori_loop(0, PCHUNK // 16, wb2, 0, unroll=8)
                            pltpu.sync_copy(
                                A.at[pl.ds(c * n + boff, PCHUNK)],
                                out_hbm.at[pl.ds(r * n + boff, PCHUNK)])
                            return carry
                        lax.fori_loop(0, n // PCHUNK, wblk, 0)

    return pl.kernel(
        body,
        out_type=jax.ShapeDtypeStruct((R * n,), jnp.float32),
        mesh=_mesh(),
        compiler_params=pltpu.CompilerParams(needs_layout_passes=False),
        scratch_types=[
            pltpu.VMEM((k_ch * nh,), jnp.float32),
            pltpu.VMEM((k_ch * n,), jnp.float32),
            pltpu.VMEM((2 * ec,), jnp.int32),
            pltpu.VMEM((2 * ec,), jnp.float32),
            pltpu.VMEM((PCHUNK,), jnp.float32),
            pltpu.VMEM((NW * 16,), jnp.int32),
            pltpu.SemaphoreType.DMA,
        ],
    )


def _apply(xcm, g, prev):
    R, n = xcm.shape
    xf = xcm.reshape(R * n)
    pf = xf if prev is None else prev.reshape(R * n)
    if g[0] == 'p':
        _, ed2, w2, cnt = g
        out = _make_apply_part(R, n, prev is None)(xf, ed2, w2, cnt, pf)
    else:
        _, ed, w = g
        out = _make_apply(R, n, prev is None)(xf, ed, w, pf)
    return out.reshape(R, n)


@functools.cache
def _make_pool(R, n):
    nh = n // 2
    CB = min(2048, nh)
    rounds = -(-R // NW)

    def body(x_hbm, out_hbm, IB, OB):
        wid = _worker_id()
        iev = 2 * lax.iota(jnp.int32, 16)
        for rnd in range(rounds):
            r = rnd * NW + wid

            @pl.when(r < R)
            def _():
                def blk(bi, carry):
                    pltpu.sync_copy(
                        x_hbm.at[pl.ds(r * n + bi * 2 * CB, 2 * CB)], IB)

                    def ibody(j, carry2):
                        base = j * 32
                        a = plsc.load_gather(IB, [iev + base])
                        b = plsc.load_gather(IB, [iev + base + 1])
                        OB[pl.ds(j * 16, 16)] = jnp.maximum(a, b)
                        return carry2
                    lax.fori_loop(0, CB // 16, ibody, 0, unroll=8)
                    pltpu.sync_copy(
                        OB, out_hbm.at[pl.ds(r * nh + bi * CB, CB)])
                    return carry
                lax.fori_loop(0, nh // CB, blk, 0)

    return pl.kernel(
        body,
        out_type=jax.ShapeDtypeStruct((R * nh,), jnp.float32),
        mesh=_mesh(),
        compiler_params=pltpu.CompilerParams(needs_layout_passes=False),
        scratch_types=[
            pltpu.VMEM((2 * CB,), jnp.float32),
            pltpu.VMEM((CB,), jnp.float32),
        ],
    )


def _pool_cm(xcm):
    R, n = xcm.shape
    return _make_pool(R, n)(xcm.reshape(R * n)).reshape(R, n // 2)


@functools.cache
def _make_combine(Cin, Cout, n, mode, Cr=0):
    NB = 512
    grid = (B, n // NB)
    tx_spec = pl.BlockSpec((Cin, NB), lambda b, j: (b, j))
    w_spec = pl.BlockSpec((Cout, 4 * Cin), lambda b, j: (0, 0))
    b_spec = pl.BlockSpec((Cout, 1), lambda b, j: (0, 0))
    res_spec = pl.BlockSpec((Cr, NB), lambda b, j: (b, j)) if Cr else None
    in_specs = [tx_spec, tx_spec, tx_spec, tx_spec, w_spec, b_spec]
    if mode == 'res_w':
        in_specs += [pl.BlockSpec((Cout, Cr), lambda b, j: (0, 0)), res_spec]
    elif mode == 'res_id':
        in_specs += [res_spec]

    def body(t0, t1, t2, t3, wt, bias, *rest):
        out = rest[-1]
        a = jnp.concatenate([t0[...], t1[...], t2[...], t3[...]], axis=0)
        h = jnp.dot(wt[...], a, preferred_element_type=jnp.float32) + bias[...]
        if mode == 'res_w':
            h = h + jnp.dot(rest[0][...], rest[1][...],
                            preferred_element_type=jnp.float32)
        elif mode == 'res_id':
            h = h + rest[0][...]
        out[...] = jnp.maximum(h, 0.0)

    return pl.pallas_call(
        body,
        grid=grid,
        in_specs=in_specs,
        out_specs=pl.BlockSpec((Cout, NB), lambda b, j: (b, j)),
        out_shape=jax.ShapeDtypeStruct((B * Cout, n), jnp.float32),
    )


def _cheb_cm(xcm, g, W, bias, mode='plain', res=None, sW=None):
    R, n = xcm.shape
    Cin = R // B
    Cout = W.shape[2]
    tx0 = xcm
    tx1 = _apply(tx0, g, None)
    tx2 = _apply(tx1, g, tx0)
    tx3 = _apply(tx2, g, tx1)
    wt = W.transpose(2, 0, 1).reshape(Cout, K * Cin)
    bb = bias[:, None]
    args = [tx0, tx1, tx2, tx3, wt, bb]
    Cr = 0
    if mode == 'res_w':
        args += [sW.T, res]
        Cr = res.shape[0] // B
    elif mode == 'res_id':
        args += [res]
        Cr = res.shape[0] // B
    return _make_combine(Cin, Cout, n, mode, Cr)(*args)


def _block_cm(xcm, p, name, g):
    h = _cheb_cm(xcm, g, p[name + 'c1_W'], p[name + 'c1_b'])
    sW = p.get(name + 's_W')
    if sW is None:
        return _cheb_cm(h, g, p[name + 'c2_W'], p[name + 'c2_b'],
                        mode='res_id', res=xcm)
    return _cheb_cm(h, g, p[name + 'c2_W'], p[name + 'c2_b'],
                    mode='res_w', res=xcm, sW=sW)


def _to_bvc(xcm):
    R, n = xcm.shape
    return xcm.reshape(B, R // B, n).transpose(0, 2, 1)


def kernel(x, src5, dst5, w5, src4, dst4, w4, src3, dst3, w3, src2, dst2, w2, src1, dst1, w1, src0, dst0, w0, conv_W, conv_b, b5c1_W, b5c1_b, b5c2_W, b5c2_b, b5s_W, b4c1_W, b4c1_b, b4c2_W, b4c2_b, b4s_W, b3c1_W, b3c1_b, b3c2_W, b3c2_b, b3s_W, b2c1_W, b2c1_b, b2c2_W, b2c2_b, b2s_W, b1c1_W, b1c1_b, b1c2_W, b1c2_b, b0c1_W, b0c1_b, b0c2_W, b0c2_b):
    kw = dict(locals())
    graphs = {}
    for i, n_lvl in enumerate(LEVELS):
        lvl = 5 - i
        ed = kw['src%d' % lvl] | (kw['dst%d' % lvl] << 16)
        w_lvl = kw['w%d' % lvl]
        if n_lvl in K_CH_P:
            ed2, w2, cnt = _make_partition(n_lvl)(ed, w_lvl)
            graphs[lvl] = ('p', ed2, w2, cnt)
        else:
            graphs[lvl] = ('d', ed, w_lvl)
    p = {k: v for k, v in kw.items() if k.endswith('_W') or k.endswith('_b')}

    x3 = x.transpose(0, 2, 1)
    x3 = jnp.pad(x3, ((0, 0), (0, 2), (0, 0)))
    xcm = x3.reshape(B * 8, x.shape[1])
    conv_Wp = jnp.pad(p['conv_W'], ((0, 0), (0, 2), (0, 0)))
    h = _cheb_cm(xcm, graphs[5], conv_Wp, p['conv_b'])
    x5 = _block_cm(h, p, 'b5', graphs[5])
    x4 = _block_cm(_pool_cm(x5), p, 'b4', graphs[4])
    x3 = _block_cm(_pool_cm(x4), p, 'b3', graphs[3])
    x2 = _block_cm(_pool_cm(x3), p, 'b2', graphs[2])
    x1 = _block_cm(_pool_cm(x2), p, 'b1', graphs[1])
    x0 = _block_cm(_pool_cm(x1), p, 'b0', graphs[0])
    return tuple(_to_bvc(v) for v in (x0, x1, x2, x3, x4, x5))
